# baseline (speedup 1.0000x reference)
import jax
import jax.numpy as jnp
from jax.experimental import pallas as pl
from jax.experimental.pallas import tpu as pltpu

_V = 21128
_B, _S = 16, 128
_ROWS = _B * _S
_R = 8
_MID = _S - 2


def _stats_kernel(sen_ref, noise_ref, x_ref, py_ref, gl_ref, out_ref):
    sen = sen_ref[...]
    noise = noise_ref[...]
    idx = jax.lax.broadcasted_iota(jnp.int32, (_R, _V), 1)

    x = x_ref[...]
    m = jnp.max(x, axis=1, keepdims=True)
    se = jnp.sum(jnp.exp(x - m), axis=1, keepdims=True)
    amax = jnp.min(jnp.where(x == m, idx, _V), axis=1, keepdims=True)
    lsen = jnp.sum(jnp.where(idx == sen, x, 0.0), axis=1, keepdims=True)
    lnoise = jnp.sum(jnp.where(idx == noise, x, 0.0), axis=1, keepdims=True)

    p = py_ref[...]
    mp = jnp.max(p, axis=1, keepdims=True)
    sep = jnp.sum(jnp.exp(p - mp), axis=1, keepdims=True)
    lsenp = jnp.sum(jnp.where(idx == sen, p, 0.0), axis=1, keepdims=True)

    g = gl_ref[...]
    mg = jnp.max(g, axis=1, keepdims=True)
    seg = jnp.sum(jnp.exp(g - mg), axis=1, keepdims=True)
    lseng = jnp.sum(jnp.where(idx == sen, g, 0.0), axis=1, keepdims=True)

    out_ref[:, 0:1] = m
    out_ref[:, 1:2] = se
    out_ref[:, 2:3] = lsen
    out_ref[:, 3:4] = lnoise
    out_ref[:, 4:5] = amax.astype(jnp.float32)
    out_ref[:, 5:6] = mp
    out_ref[:, 6:7] = sep
    out_ref[:, 7:8] = lsenp
    out_ref[:, 8:9] = mg
    out_ref[:, 9:10] = seg
    out_ref[:, 10:11] = lseng
    out_ref[:, 11:16] = jnp.zeros((_R, 5), jnp.float32)


def _prf_block(TP, TN, FP):
    eps = 1e-8
    P = TP / (TP + FP + eps)
    R = TP / (TP + TN + eps)
    F = 2.0 * P * R / (P + R + eps)
    return P, R, F


def _epi_kernel(sen_ref, noise_ref, mask_ref, thresh_ref, threshup_ref,
                m_ref, se_ref, lsen_ref, lnoise_ref, amax_ref,
                mp_ref, sep_ref, lsenp_ref, mg_ref, seg_ref, lseng_ref,
                loss_ref, acc_ref, ratio_ref, e0_ref, e_ref, mets_ref):
    sen = sen_ref[...]
    noise = noise_ref[...]
    maskf = mask_ref[...]
    maskb = maskf != 0.0
    t = thresh_ref[...]
    tu = threshup_ref[...]

    m = m_ref[...]
    ce = jnp.log(se_ref[...]) + m - lsen_ref[...]
    cep = jnp.log(sep_ref[...]) + mp_ref[...] - lsenp_ref[...]
    ceg = jnp.log(seg_ref[...]) + mg_ref[...] - lseng_ref[...]
    loss_ref[...] = jnp.sum(jnp.where(maskb, ce + cep + ceg, 0.0),
                            keepdims=True).reshape(1, 1)

    amax = amax_ref[...].astype(jnp.int32)
    pred = jnp.where(maskb, amax, 0)
    correct = jnp.where(maskb, (pred == sen).astype(jnp.float32), 0.0)
    acc_ref[...] = (jnp.sum(correct, keepdims=True).reshape(1, 1)
                    / jnp.maximum(jnp.sum(maskf, keepdims=True).reshape(1, 1),
                                  1.0))

    m_mid = m[:, 1:_S - 1]
    lnoise_mid = lnoise_ref[...][:, 1:_S - 1]
    ratio = jnp.exp(m_mid - lnoise_mid)
    e0b = ratio > tu
    eb = jnp.logical_and(ratio < t, jnp.logical_not(e0b))
    noise_mid = noise[:, 1:_S - 1]
    china = jnp.logical_and(noise_mid > 670, noise_mid < 7992)
    nchina = jnp.logical_not(china)
    e0_out = jnp.logical_or(jnp.logical_not(e0b), nchina)
    eb = jnp.logical_or(eb, nchina)
    ratio_ref[...] = jnp.where(eb, 1.0, ratio)
    e0_ref[...] = e0_out.astype(jnp.int32)
    e_ref[...] = eb.astype(jnp.int32)

    sen_mid = sen[:, 1:_S - 1]
    amax_mid = amax[:, 1:_S - 1]
    topone = jnp.where(eb, sen_mid, amax_mid)
    bl = noise_mid == sen_mid
    nbl = jnp.logical_not(bl)
    nerr = jnp.logical_not(eb)

    def _s(v):
        return jnp.sum(v.astype(jnp.float32), keepdims=True).reshape(1, 1)

    tpd = jnp.logical_and(nbl, nerr)
    tnd = jnp.logical_and(nbl, eb)
    fpd = jnp.logical_and(bl, nerr)
    TPD, TND, FPD = _s(tpd), _s(tnd), _s(fpd)

    t1 = topone == sen_mid
    tpc = jnp.logical_and(tpd, t1)
    tnc = jnp.logical_or(tnd, jnp.logical_and(tpd, jnp.logical_not(t1)))
    TPC, TNC, FPC = _s(tpc), _s(tnc), FPD

    bl_i = 1 - bl.astype(jnp.int32)
    err2 = 1 - eb.astype(jnp.int32)
    binlabelsum = jnp.sum(bl_i, axis=1, keepdims=True)
    lmes = jnp.sum(jnp.abs(bl_i - err2), axis=1, keepdims=True)
    haspos = binlabelsum > 0
    tpsd = jnp.logical_and(haspos, lmes == 0)
    tnsd = jnp.logical_and(haspos, lmes > 0)
    fpsd = jnp.logical_and(binlabelsum == 0, lmes > 0)
    TPSD, TNSD, FPSD = _s(tpsd), _s(tnsd), _s(fpsd)

    toponesen = jnp.sum(jnp.logical_not(t1).astype(jnp.int32), axis=1,
                        keepdims=True) == 0
    tpsc = jnp.logical_and(tpsd, toponesen)
    tnsc = jnp.logical_and(
        haspos,
        jnp.logical_or(lmes > 0,
                       jnp.logical_and(lmes == 0,
                                       jnp.logical_not(toponesen))))
    TPSC, TNSC, FPSC = _s(tpsc), _s(tnsc), FPSD

    PD, RD, FD = _prf_block(TPD, TND, FPD)
    PC, RC, FC = _prf_block(TPC, TNC, FPC)
    PSD, RSD, FSD = _prf_block(TPSD, TNSD, FPSD)
    PSC, RSC, FSC = _prf_block(TPSC, TNSC, FPSC)

    mets_ref[...] = jnp.concatenate(
        [TPD, TND, FPD, TPC, TNC, FPC, TPSD, TNSD, FPSD, TPSC, TNSC, FPSC,
         PD, RD, FD, PC, RC, FC, PSD, RSD, FSD, PSC, RSC, FSC], axis=1)


def kernel(sen, noise, logits, logitspy, logitsglyph, sequence_mask, sumls,
           pri, thresh, threshup):
    x = logits.reshape(_ROWS, _V)
    p = logitspy.reshape(_ROWS, _V)
    g = logitsglyph.reshape(_ROWS, _V)
    sen2 = sen.reshape(_ROWS, 1)
    noise2 = noise.reshape(_ROWS, 1)

    stats = pl.pallas_call(
        _stats_kernel,
        grid=(_ROWS // _R,),
        in_specs=[
            pl.BlockSpec((_R, 1), lambda i: (i, 0)),
            pl.BlockSpec((_R, 1), lambda i: (i, 0)),
            pl.BlockSpec((_R, _V), lambda i: (i, 0)),
            pl.BlockSpec((_R, _V), lambda i: (i, 0)),
            pl.BlockSpec((_R, _V), lambda i: (i, 0)),
        ],
        out_specs=pl.BlockSpec((_R, 16), lambda i: (i, 0)),
        out_shape=jax.ShapeDtypeStruct((_ROWS, 16), jnp.float32),
        compiler_params=pltpu.CompilerParams(
            dimension_semantics=("arbitrary",)),
    )(sen2, noise2, x, p, g)

    st = stats.reshape(_B, _S, 16)
    m, se, lsen, lnoise, amaxf = (st[..., 0], st[..., 1], st[..., 2],
                                  st[..., 3], st[..., 4])
    mp, sep, lsenp = st[..., 5], st[..., 6], st[..., 7]
    mg, seg, lseng = st[..., 8], st[..., 9], st[..., 10]

    maskf = sequence_mask.astype(jnp.float32)
    tarr = jnp.asarray(thresh, jnp.float32).reshape(1, 1)
    tuarr = jnp.asarray(threshup, jnp.float32).reshape(1, 1)

    loss_a, acc_a, ratio, e0, e, mets = pl.pallas_call(
        _epi_kernel,
        out_shape=[
            jax.ShapeDtypeStruct((1, 1), jnp.float32),
            jax.ShapeDtypeStruct((1, 1), jnp.float32),
            jax.ShapeDtypeStruct((_B, _MID), jnp.float32),
            jax.ShapeDtypeStruct((_B, _MID), jnp.int32),
            jax.ShapeDtypeStruct((_B, _MID), jnp.int32),
            jax.ShapeDtypeStruct((1, 24), jnp.float32),
        ],
    )(sen, noise, maskf, tarr, tuarr, m, se, lsen, lnoise, amaxf,
      mp, sep, lsenp, mg, seg, lseng)

    loss = loss_a[0, 0]
    acc = acc_a[0, 0]
    ms = tuple(mets[0, i] for i in range(24))
    return (loss, acc, jnp.asarray(sumls, jnp.float32), ratio, e0, e) + ms
